# baseline (speedup 1.0000x reference)
import jax
import jax.numpy as jnp
from jax import lax
from jax.experimental import pallas as pl
from jax.experimental.pallas import tpu as pltpu
from jax.experimental.pallas import tpu_sc as plsc

N = 10000
E = 160000
DIN = 256
HID = 512
FC = 128
NP = 10240
NS = 16
NC = 2
B = 128
NB = 80
EP = NS * NB * B
NBUF = 2
SB = 16
SEG = NB // SB
RPT = NP // NS
MT = 512
NCH1 = DIN // FC
NCH2 = HID // FC

_MESH = plsc.VectorSubcoreMesh(
    core_axis_name="c", subcore_axis_name="s", num_cores=NC, num_subcores=NS
)


def _deg_body(dst1, dst2, dinv1, dinv2, hist_v, idx_v, red_v, out_v, shared):
    c = lax.axis_index("c")
    s = lax.axis_index("s")

    def run(dst_ref, out_ref):
        def zero(i, carry):
            hist_v[pl.ds(i * 16, 16)] = jnp.zeros((16,), jnp.float32)
            return carry

        lax.fori_loop(0, NP // 16, zero, 0)

        ones = jnp.ones((16,), jnp.float32)
        pltpu.sync_copy(dst_ref.at[s], idx_v)

        def batch(b, carry):
            for j in range(B // 16):
                idx16 = idx_v[b, pl.ds(j * 16, 16)]
                plsc.addupdate_scatter(hist_v, [idx16], ones)
            return carry

        lax.fori_loop(0, NB, batch, 0)

        pltpu.sync_copy(hist_v, shared.at[s])
        plsc.subcore_barrier()
        pltpu.sync_copy(shared.at[:, pl.ds(s * RPT, RPT)], red_v)

        def col(j, carry):
            acc = red_v[0, pl.ds(j * 16, 16)]
            for r in range(1, NS):
                acc = acc + red_v[r, pl.ds(j * 16, 16)]
            d = acc + 1.0
            yi = jnp.full((16,), 0x5F3759DF, jnp.int32) - lax.shift_right_logical(
                plsc.bitcast(d, jnp.int32), 1
            )
            y = plsc.bitcast(yi, jnp.float32)
            for _ in range(3):
                y = y * (1.5 - 0.5 * d * y * y)
            out_v[pl.ds(j * 16, 16)] = y
            return carry

        lax.fori_loop(0, RPT // 16, col, 0)
        pltpu.sync_copy(out_v, out_ref.at[pl.ds(s * RPT, RPT)])

    @pl.when(c == 0)
    def _():
        run(dst1, dinv1)

    @pl.when(c == 1)
    def _():
        run(dst2, dinv2)


_deg_kernel = pl.kernel(
    _deg_body,
    out_type=[jax.ShapeDtypeStruct((NP,), jnp.float32)] * 2,
    mesh=_MESH,
    compiler_params=pltpu.CompilerParams(needs_layout_passes=False),
    scratch_types=[
        pltpu.VMEM((NP,), jnp.float32),
        pltpu.VMEM((NB, B), jnp.int32),
        pltpu.VMEM((NS, RPT), jnp.float32),
        pltpu.VMEM((RPT,), jnp.float32),
        pltpu.VMEM_SHARED((NS, NP), jnp.float32),
    ],
)


def _make_agg_body(npc):
    nch = npc * NC

    def body(*refs):
        xsf = refs[0:nch]
        xb = refs[nch : 2 * nch]
        srcp = refs[2 * nch]
        dstp = refs[2 * nch + 1]
        outs = refs[2 * nch + 2 : 3 * nch + 2]
        rest = refs[3 * nch + 2 :]
        acc, src_buf, dst_buf = rest[0:3]
        rows_bf = rest[3 : 3 + NBUF]
        rows_f = rest[3 + NBUF]
        sg = rest[4 + NBUF : 4 + 2 * NBUF]
        c = lax.axis_index("c")
        s = lax.axis_index("s")

        def proc(xsf_ref, xb_ref, out_ref):
            pltpu.sync_copy(
                xsf_ref.at[pl.ds(s * RPT, RPT)], acc.at[pl.ds(s * RPT, RPT)]
            )
            plsc.subcore_barrier()

            def start_gather(j, b):
                pltpu.async_copy(xb_ref.at[src_buf.at[b]], rows_bf[j], sg[j])

            def wait_gather(j, b):
                pltpu.make_async_copy(
                    xb_ref.at[src_buf.at[b]], rows_bf[j], sg[j]
                ).wait()

            def convert(j):
                def row(i, carry):
                    for w in range(FC // 32):
                        grp = rows_bf[j][i, pl.ds(w * 32, 32)]
                        a, b2 = plsc.unpack(
                            grp, format=plsc.PackFormat.INTERLEAVED
                        )
                        rows_f[i, pl.ds(w * 32, 16)] = a
                        rows_f[i, pl.ds(w * 32 + 16, 16)] = b2
                    return carry

                lax.fori_loop(0, B, row, 0)

            def do_batch(j, b):
                wait_gather(j, b)
                convert(j)
                pltpu.sync_copy(rows_f, acc.at[dst_buf.at[b]], add=True)
                return None

            def segment(w, carry):
                pltpu.sync_copy(srcp.at[s, pl.ds(w * SB, SB)], src_buf)
                pltpu.sync_copy(dstp.at[s, pl.ds(w * SB, SB)], dst_buf)
                for j in range(NBUF):
                    start_gather(j, j)

                def round_(g, carry2):
                    b0 = g * NBUF
                    for j in range(NBUF):
                        do_batch(j, b0 + j)
                        start_gather(j, b0 + NBUF + j)
                    return carry2

                lax.fori_loop(0, SB // NBUF - 1, round_, 0)

                b0 = SB - NBUF
                for j in range(NBUF):
                    do_batch(j, b0 + j)
                return carry

            lax.fori_loop(0, SEG, segment, 0)

            plsc.subcore_barrier()
            pltpu.sync_copy(
                acc.at[pl.ds(s * RPT, RPT)], out_ref.at[pl.ds(s * RPT, RPT)]
            )
            plsc.subcore_barrier()

        for k in range(npc):

            @pl.when(c == 0)
            def _():
                proc(xsf[k], xb[k], outs[k])

            @pl.when(c == 1)
            def _():
                proc(xsf[npc + k], xb[npc + k], outs[npc + k])

    return body


def _make_agg_kernel(npc):
    nch = npc * NC
    return pl.kernel(
        _make_agg_body(npc),
        out_type=[jax.ShapeDtypeStruct((NP, FC), jnp.float32)] * nch,
        mesh=_MESH,
        compiler_params=pltpu.CompilerParams(
            use_tc_tiling_on_sc=False, needs_layout_passes=False
        ),
        scratch_types=[
            pltpu.VMEM_SHARED((NP, FC), jnp.float32),
            pltpu.VMEM((SB, B), jnp.int32),
            pltpu.VMEM((SB, B), jnp.int32),
        ]
        + [pltpu.VMEM((B, FC), jnp.bfloat16)] * NBUF
        + [pltpu.VMEM((B, FC), jnp.float32)]
        + [pltpu.SemaphoreType.DMA] * NBUF,
    )


_agg1 = _make_agg_kernel(1)
_agg2 = _make_agg_kernel(2)


def _pair_shuffle_bf16(h):
    m, w = h.shape
    u = h.reshape(m, w // 32, 2, 16)
    v = jnp.swapaxes(u, 2, 3)
    return v.reshape(m, w).astype(jnp.bfloat16)


def _scale_kernel_body(x_ref, dv_ref, *o_refs):
    xs = x_ref[...] * dv_ref[...]
    for k in range(NCH1):
        chunk = xs[:, k * FC : (k + 1) * FC]
        o_refs[k][...] = chunk
        o_refs[NCH1 + k][...] = _pair_shuffle_bf16(chunk)


@jax.jit
def _scale(x_pad, dv):
    return pl.pallas_call(
        _scale_kernel_body,
        grid=(NP // MT,),
        in_specs=[
            pl.BlockSpec((MT, DIN), lambda i: (i, 0)),
            pl.BlockSpec((MT, 1), lambda i: (i, 0)),
        ],
        out_specs=[pl.BlockSpec((MT, FC), lambda i: (i, 0))] * (2 * NCH1),
        out_shape=[jax.ShapeDtypeStruct((NP, FC), jnp.float32)] * NCH1
        + [jax.ShapeDtypeStruct((NP, FC), jnp.bfloat16)] * NCH1,
    )(x_pad, dv)


def _mm1_body(*refs):
    s_refs = refs[0:NCH1]
    w_ref, b_ref, dv_ref = refs[NCH1 : NCH1 + 3]
    o_refs = refs[NCH1 + 3 :]
    acc = jnp.dot(
        s_refs[0][...], w_ref[:FC, :], preferred_element_type=jnp.float32
    )
    for k in range(1, NCH1):
        acc = acc + jnp.dot(
            s_refs[k][...],
            w_ref[k * FC : (k + 1) * FC, :],
            preferred_element_type=jnp.float32,
        )
    dv = dv_ref[...]
    h = jnp.maximum(acc * dv + b_ref[...], 0.0) * dv
    for k in range(NCH2):
        chunk = h[:, k * FC : (k + 1) * FC]
        o_refs[k][...] = chunk
        o_refs[NCH2 + k][...] = _pair_shuffle_bf16(chunk)


@jax.jit
def _mm1(s_chunks, w1, b1, dv):
    return pl.pallas_call(
        _mm1_body,
        grid=(NP // MT,),
        in_specs=[pl.BlockSpec((MT, FC), lambda i: (i, 0))] * NCH1
        + [
            pl.BlockSpec((DIN, HID), lambda i: (0, 0)),
            pl.BlockSpec((1, HID), lambda i: (0, 0)),
            pl.BlockSpec((MT, 1), lambda i: (i, 0)),
        ],
        out_specs=[pl.BlockSpec((MT, FC), lambda i: (i, 0))] * (2 * NCH2),
        out_shape=[jax.ShapeDtypeStruct((NP, FC), jnp.float32)] * NCH2
        + [jax.ShapeDtypeStruct((NP, FC), jnp.bfloat16)] * NCH2,
    )(*s_chunks, w1, b1, dv)


def _mm2_body(*refs):
    s_refs = refs[0:NCH2]
    w_ref, b_ref, dv_ref = refs[NCH2 : NCH2 + 3]
    o_ref = refs[NCH2 + 3]
    acc = jnp.dot(
        s_refs[0][...], w_ref[:FC, :], preferred_element_type=jnp.float32
    )
    for k in range(1, NCH2):
        acc = acc + jnp.dot(
            s_refs[k][...],
            w_ref[k * FC : (k + 1) * FC, :],
            preferred_element_type=jnp.float32,
        )
    o_ref[...] = acc * dv_ref[...] + b_ref[...]


@jax.jit
def _mm2(s_chunks, w2, b2, dv):
    return pl.pallas_call(
        _mm2_body,
        grid=(NP // MT,),
        in_specs=[pl.BlockSpec((MT, FC), lambda i: (i, 0))] * NCH2
        + [
            pl.BlockSpec((HID, HID), lambda i: (0, 0)),
            pl.BlockSpec((1, HID), lambda i: (0, 0)),
            pl.BlockSpec((MT, 1), lambda i: (i, 0)),
        ],
        out_specs=pl.BlockSpec((MT, HID), lambda i: (i, 0)),
        out_shape=jax.ShapeDtypeStruct((NP, HID), jnp.float32),
    )(*s_chunks, w2, b2, dv)


def _view(x, srcp, dstp, dv, W1, b1r, W2, b2r):
    x_pad = jnp.pad(x, ((0, NP - N), (0, 0)))
    xs = _scale(x_pad, dv)
    s = _agg1(*xs, srcp, dstp)
    h = _mm1(s, W1, b1r, dv)
    t = _agg2(*h, srcp, dstp)
    z = _mm2(t, W2, b2r, dv)
    return z[:N]


def kernel(x1, edge_index1, x2, edge_index2, W1, b1, W2, b2):
    pad_src = jnp.zeros((EP - E,), jnp.int32)
    pad_dst = jnp.full((EP - E,), N, jnp.int32)
    src1 = jnp.concatenate([edge_index1[0], pad_src]).reshape(NS, NB, B)
    dst1 = jnp.concatenate([edge_index1[1], pad_dst]).reshape(NS, NB, B)
    src2 = jnp.concatenate([edge_index2[0], pad_src]).reshape(NS, NB, B)
    dst2 = jnp.concatenate([edge_index2[1], pad_dst]).reshape(NS, NB, B)

    dinv1, dinv2 = _deg_kernel(dst1, dst2)
    dv1 = dinv1.reshape(NP, 1)
    dv2 = dinv2.reshape(NP, 1)
    b1r = b1.reshape(1, HID)
    b2r = b2.reshape(1, HID)

    z1 = _view(x1, src1, dst1, dv1, W1, b1r, W2, b2r)
    z2 = _view(x2, src2, dst2, dv2, W1, b1r, W2, b2r)
    return (z1, z2)


# speedup vs baseline: 1.5201x; 1.5201x over previous
import jax
import jax.numpy as jnp
from jax import lax
from jax.experimental import pallas as pl
from jax.experimental.pallas import tpu as pltpu
from jax.experimental.pallas import tpu_sc as plsc

N = 10000
E = 160000
DIN = 256
HID = 512
FC = 128
NP = 10240
NS = 16
NC = 2
B = 80
NB = 128
EP = NS * NB * B
NBUF = 4
SB = 32
SEG = NB // SB
RPT = NP // NS
MT = 512
NCH1 = DIN // FC
NCH2 = HID // FC

_MESH = plsc.VectorSubcoreMesh(
    core_axis_name="c", subcore_axis_name="s", num_cores=NC, num_subcores=NS
)


def _deg_body(dst1, dst2, dinv1, dinv2, hist_v, idx_v, red_v, out_v, shared):
    c = lax.axis_index("c")
    s = lax.axis_index("s")

    def run(dst_ref, out_ref):
        def zero(i, carry):
            hist_v[pl.ds(i * 16, 16)] = jnp.zeros((16,), jnp.float32)
            return carry

        lax.fori_loop(0, NP // 16, zero, 0)

        ones = jnp.ones((16,), jnp.float32)
        pltpu.sync_copy(dst_ref.at[s], idx_v)

        def batch(b, carry):
            for j in range(B // 16):
                idx16 = idx_v[b, pl.ds(j * 16, 16)]
                plsc.addupdate_scatter(hist_v, [idx16], ones)
            return carry

        lax.fori_loop(0, NB, batch, 0)

        pltpu.sync_copy(hist_v, shared.at[s])
        plsc.subcore_barrier()
        pltpu.sync_copy(shared.at[:, pl.ds(s * RPT, RPT)], red_v)

        def col(j, carry):
            acc = red_v[0, pl.ds(j * 16, 16)]
            for r in range(1, NS):
                acc = acc + red_v[r, pl.ds(j * 16, 16)]
            d = acc + 1.0
            yi = jnp.full((16,), 0x5F3759DF, jnp.int32) - lax.shift_right_logical(
                plsc.bitcast(d, jnp.int32), 1
            )
            y = plsc.bitcast(yi, jnp.float32)
            for _ in range(3):
                y = y * (1.5 - 0.5 * d * y * y)
            out_v[pl.ds(j * 16, 16)] = y
            return carry

        lax.fori_loop(0, RPT // 16, col, 0)
        pltpu.sync_copy(out_v, out_ref.at[pl.ds(s * RPT, RPT)])

    @pl.when(c == 0)
    def _():
        run(dst1, dinv1)

    @pl.when(c == 1)
    def _():
        run(dst2, dinv2)


_deg_kernel = pl.kernel(
    _deg_body,
    out_type=[jax.ShapeDtypeStruct((NP,), jnp.float32)] * 2,
    mesh=_MESH,
    compiler_params=pltpu.CompilerParams(needs_layout_passes=False),
    scratch_types=[
        pltpu.VMEM((NP,), jnp.float32),
        pltpu.VMEM((NB, B), jnp.int32),
        pltpu.VMEM((NS, RPT), jnp.float32),
        pltpu.VMEM((RPT,), jnp.float32),
        pltpu.VMEM_SHARED((NS, NP), jnp.float32),
    ],
)


def _make_agg_body(npc):
    nch = npc * NC

    def body(*refs):
        xsf = refs[0:nch]
        xb = refs[nch : 2 * nch]
        srcp = refs[2 * nch]
        dstp = refs[2 * nch + 1]
        outs = refs[2 * nch + 2 : 3 * nch + 2]
        rest = refs[3 * nch + 2 :]
        acc, src_buf, dst_buf = rest[0:3]
        rows_bf = rest[3 : 3 + NBUF]
        rows_f = rest[3 + NBUF]
        sg = rest[4 + NBUF : 4 + 2 * NBUF]
        c = lax.axis_index("c")
        s = lax.axis_index("s")

        def proc(xsf_ref, xb_ref, out_ref):
            pltpu.sync_copy(
                xsf_ref.at[pl.ds(s * RPT, RPT)], acc.at[pl.ds(s * RPT, RPT)]
            )
            plsc.subcore_barrier()

            def start_gather(j, b):
                pltpu.async_copy(xb_ref.at[src_buf.at[b]], rows_bf[j], sg[j])

            def wait_gather(j, b):
                pltpu.make_async_copy(
                    xb_ref.at[src_buf.at[b]], rows_bf[j], sg[j]
                ).wait()

            def convert(j):
                def row(i2, carry):
                    for u in range(2):
                        i = i2 * 2 + u
                        for w in range(FC // 32):
                            grp = rows_bf[j][i, pl.ds(w * 32, 32)]
                            a, b2 = plsc.unpack(
                                grp, format=plsc.PackFormat.INTERLEAVED
                            )
                            rows_f[i, pl.ds(w * 32, 16)] = a
                            rows_f[i, pl.ds(w * 32 + 16, 16)] = b2
                    return carry

                lax.fori_loop(0, B // 2, row, 0)

            def do_batch(j, b):
                wait_gather(j, b)
                convert(j)
                pltpu.sync_copy(rows_f, acc.at[dst_buf.at[b]], add=True)
                return None

            def segment(w, carry):
                pltpu.sync_copy(srcp.at[s, pl.ds(w * SB, SB)], src_buf)
                pltpu.sync_copy(dstp.at[s, pl.ds(w * SB, SB)], dst_buf)
                for j in range(NBUF):
                    start_gather(j, j)

                def round_(g, carry2):
                    b0 = g * NBUF
                    for j in range(NBUF):
                        do_batch(j, b0 + j)
                        start_gather(j, b0 + NBUF + j)
                    return carry2

                lax.fori_loop(0, SB // NBUF - 1, round_, 0)

                b0 = SB - NBUF
                for j in range(NBUF):
                    do_batch(j, b0 + j)
                return carry

            lax.fori_loop(0, SEG, segment, 0)

            plsc.subcore_barrier()
            pltpu.sync_copy(
                acc.at[pl.ds(s * RPT, RPT)], out_ref.at[pl.ds(s * RPT, RPT)]
            )
            plsc.subcore_barrier()

        for k in range(npc):

            @pl.when(c == 0)
            def _():
                proc(xsf[k], xb[k], outs[k])

            @pl.when(c == 1)
            def _():
                proc(xsf[npc + k], xb[npc + k], outs[npc + k])

    return body


def _make_agg_kernel(npc):
    nch = npc * NC
    return pl.kernel(
        _make_agg_body(npc),
        out_type=[jax.ShapeDtypeStruct((NP, FC), jnp.float32)] * nch,
        mesh=_MESH,
        compiler_params=pltpu.CompilerParams(
            use_tc_tiling_on_sc=False, needs_layout_passes=False
        ),
        scratch_types=[
            pltpu.VMEM_SHARED((NP, FC), jnp.float32),
            pltpu.VMEM((SB, B), jnp.int32),
            pltpu.VMEM((SB, B), jnp.int32),
        ]
        + [pltpu.VMEM((B, FC), jnp.bfloat16)] * NBUF
        + [pltpu.VMEM((B, FC), jnp.float32)]
        + [pltpu.SemaphoreType.DMA] * NBUF,
    )


_agg1 = _make_agg_kernel(1)
_agg2 = _make_agg_kernel(2)


def _perm_matrix():
    import numpy as _np

    perm = _np.zeros((FC,), _np.int64)
    for g in range(FC // 32):
        for p_ in range(2):
            for i in range(16):
                perm[32 * g + 16 * p_ + i] = 32 * g + 2 * i + p_
    mat = _np.zeros((FC, FC), _np.float32)
    mat[_np.arange(FC), perm] = 1.0
    return mat


_PERM = _perm_matrix()


def _scale_kernel_body(x_ref, dv_ref, pm_ref, *o_refs):
    xs = x_ref[...] * dv_ref[...]
    pm = pm_ref[...]
    for k in range(NCH1):
        chunk = xs[:, k * FC : (k + 1) * FC]
        o_refs[k][...] = chunk
        o_refs[NCH1 + k][...] = jnp.dot(
            chunk, pm, preferred_element_type=jnp.float32
        ).astype(jnp.bfloat16)


@jax.jit
def _scale(x_pad, dv, pm):
    return pl.pallas_call(
        _scale_kernel_body,
        grid=(NP // MT,),
        in_specs=[
            pl.BlockSpec((MT, DIN), lambda i: (i, 0)),
            pl.BlockSpec((MT, 1), lambda i: (i, 0)),
            pl.BlockSpec((FC, FC), lambda i: (0, 0)),
        ],
        out_specs=[pl.BlockSpec((MT, FC), lambda i: (i, 0))] * (2 * NCH1),
        out_shape=[jax.ShapeDtypeStruct((NP, FC), jnp.float32)] * NCH1
        + [jax.ShapeDtypeStruct((NP, FC), jnp.bfloat16)] * NCH1,
    )(x_pad, dv, pm)


def _mm1_body(*refs):
    s_refs = refs[0:NCH1]
    w_ref, b_ref, dv_ref, pm_ref = refs[NCH1 : NCH1 + 4]
    o_refs = refs[NCH1 + 4 :]
    acc = jnp.dot(
        s_refs[0][...], w_ref[:FC, :], preferred_element_type=jnp.float32
    )
    for k in range(1, NCH1):
        acc = acc + jnp.dot(
            s_refs[k][...],
            w_ref[k * FC : (k + 1) * FC, :],
            preferred_element_type=jnp.float32,
        )
    dv = dv_ref[...]
    pm = pm_ref[...]
    h = jnp.maximum(acc * dv + b_ref[...], 0.0) * dv
    for k in range(NCH2):
        chunk = h[:, k * FC : (k + 1) * FC]
        o_refs[k][...] = chunk
        o_refs[NCH2 + k][...] = jnp.dot(
            chunk, pm, preferred_element_type=jnp.float32
        ).astype(jnp.bfloat16)


@jax.jit
def _mm1(s_chunks, w1, b1, dv, pm):
    return pl.pallas_call(
        _mm1_body,
        grid=(NP // MT,),
        in_specs=[pl.BlockSpec((MT, FC), lambda i: (i, 0))] * NCH1
        + [
            pl.BlockSpec((DIN, HID), lambda i: (0, 0)),
            pl.BlockSpec((1, HID), lambda i: (0, 0)),
            pl.BlockSpec((MT, 1), lambda i: (i, 0)),
            pl.BlockSpec((FC, FC), lambda i: (0, 0)),
        ],
        out_specs=[pl.BlockSpec((MT, FC), lambda i: (i, 0))] * (2 * NCH2),
        out_shape=[jax.ShapeDtypeStruct((NP, FC), jnp.float32)] * NCH2
        + [jax.ShapeDtypeStruct((NP, FC), jnp.bfloat16)] * NCH2,
    )(*s_chunks, w1, b1, dv, pm)


def _mm2_body(*refs):
    s_refs = refs[0:NCH2]
    w_ref, b_ref, dv_ref = refs[NCH2 : NCH2 + 3]
    o_ref = refs[NCH2 + 3]
    acc = jnp.dot(
        s_refs[0][...], w_ref[:FC, :], preferred_element_type=jnp.float32
    )
    for k in range(1, NCH2):
        acc = acc + jnp.dot(
            s_refs[k][...],
            w_ref[k * FC : (k + 1) * FC, :],
            preferred_element_type=jnp.float32,
        )
    o_ref[...] = acc * dv_ref[...] + b_ref[...]


@jax.jit
def _mm2(s_chunks, w2, b2, dv):
    return pl.pallas_call(
        _mm2_body,
        grid=(NP // MT,),
        in_specs=[pl.BlockSpec((MT, FC), lambda i: (i, 0))] * NCH2
        + [
            pl.BlockSpec((HID, HID), lambda i: (0, 0)),
            pl.BlockSpec((1, HID), lambda i: (0, 0)),
            pl.BlockSpec((MT, 1), lambda i: (i, 0)),
        ],
        out_specs=pl.BlockSpec((MT, HID), lambda i: (i, 0)),
        out_shape=jax.ShapeDtypeStruct((NP, HID), jnp.float32),
    )(*s_chunks, w2, b2, dv)


def _view(x, srcp, dstp, dv, pm, W1, b1r, W2, b2r):
    x_pad = jnp.pad(x, ((0, NP - N), (0, 0)))
    xs = _scale(x_pad, dv, pm)
    s = _agg1(*xs, srcp, dstp)
    h = _mm1(s, W1, b1r, dv, pm)
    t = _agg2(*h, srcp, dstp)
    z = _mm2(t, W2, b2r, dv)
    return z[:N]


def kernel(x1, edge_index1, x2, edge_index2, W1, b1, W2, b2):
    pad_src = jnp.zeros((EP - E,), jnp.int32)
    pad_dst = jnp.full((EP - E,), N, jnp.int32)
    src1 = jnp.concatenate([edge_index1[0], pad_src]).reshape(NS, NB, B)
    dst1 = jnp.concatenate([edge_index1[1], pad_dst]).reshape(NS, NB, B)
    src2 = jnp.concatenate([edge_index2[0], pad_src]).reshape(NS, NB, B)
    dst2 = jnp.concatenate([edge_index2[1], pad_dst]).reshape(NS, NB, B)

    dinv1, dinv2 = _deg_kernel(dst1, dst2)
    dv1 = dinv1.reshape(NP, 1)
    dv2 = dinv2.reshape(NP, 1)
    b1r = b1.reshape(1, HID)
    b2r = b2.reshape(1, HID)

    pm = jnp.asarray(_PERM)
    z1 = _view(x1, src1, dst1, dv1, pm, W1, b1r, W2, b2r)
    z2 = _view(x2, src2, dst2, dv2, pm, W1, b1r, W2, b2r)
    return (z1, z2)
